# baseline (speedup 1.0000x reference)
import functools

import jax
import jax.numpy as jnp
from jax import lax
from jax.experimental import pallas as pl
from jax.experimental.pallas import tpu as pltpu
from jax.experimental.pallas import tpu_sc as plsc

P = 0.5

_info = plsc.get_sparse_core_info()
_NC, _NS, _L = _info.num_cores, _info.num_subcores, _info.num_lanes
_NW = _NC * _NS


def _make_kernel(bn):
    ppw = bn // _NW
    cpts = 8192
    nchunk = ppw // cpts
    spans = cpts // _L
    mesh = plsc.VectorSubcoreMesh(core_axis_name="c", subcore_axis_name="s")

    @functools.partial(
        pl.kernel,
        mesh=mesh,
        out_type=jax.ShapeDtypeStruct((bn * 9,), jnp.float32),
        scratch_types=[
            pltpu.VMEM((cpts * 9,), jnp.float32),
            pltpu.VMEM((cpts,), jnp.float32),
            pltpu.VMEM((144,), jnp.float32),
            pltpu.VMEM((144,), jnp.int32),
        ],
    )
    def k(cov_hbm, flip_hbm, droppat_hbm, idxpat_hbm, out_hbm,
          cov_v, flip_v, droppat_v, idxpat_v):
        wid = lax.axis_index("s") * _NC + lax.axis_index("c")

        pltpu.sync_copy(droppat_hbm, droppat_v)
        pltpu.sync_copy(idxpat_hbm, idxpat_v)

        dropv = [droppat_v[pl.ds(16 * ph, 16)] for ph in range(9)]
        idxv = [idxpat_v[pl.ds(16 * ph, 16)] for ph in range(9)]

        def span_body(s, _):
            pbase = s * _L
            ebase = s * (_L * 9)
            f = flip_v[pl.ds(pbase, 16)]
            for ph in range(9):
                fv = lax.gather(
                    f, idxv[ph][:, None],
                    lax.GatherDimensionNumbers(
                        offset_dims=(), collapsed_slice_dims=(0,),
                        start_index_map=(0,)),
                    slice_sizes=(1,),
                    mode=lax.GatherScatterMode.PROMISE_IN_BOUNDS)
                cv = cov_v[pl.ds(ebase + 16 * ph, 16)]
                cov_v[pl.ds(ebase + 16 * ph, 16)] = jnp.where(
                    fv >= P, cv, dropv[ph])
            return 0

        def chunk_body(c, _):
            base_pt = wid * ppw + c * cpts
            pltpu.sync_copy(flip_hbm.at[pl.ds(base_pt, cpts)], flip_v)
            pltpu.sync_copy(cov_hbm.at[pl.ds(base_pt * 9, cpts * 9)], cov_v)
            lax.fori_loop(0, spans, span_body, 0)
            pltpu.sync_copy(cov_v, out_hbm.at[pl.ds(base_pt * 9, cpts * 9)])
            return 0

        lax.fori_loop(0, nchunk, chunk_body, 0)

    return k


def kernel(cov, drop_cov, flip):
    b, n, d, _ = cov.shape
    bn = b * n
    cov_flat = cov.reshape(bn * d * d)
    j = jnp.arange(144, dtype=jnp.int32)
    idx_pat = j // 9
    drop_pat = drop_cov.reshape(9)[j % 9]
    out = _make_kernel(bn)(cov_flat, flip, drop_pat, idx_pat)
    return out.reshape(b, n, d, d)
